# baseline (speedup 1.0000x reference)
import dataclasses
import functools

import jax
import jax.numpy as jnp
from jax import lax
from jax.experimental import pallas as pl
from jax.experimental.pallas import tpu as pltpu
from jax.experimental.pallas import tpu_sc as plsc

TOKENS = 16384
D = 1024
K = 8
L = 16
NW = 32
TPW = TOKENS // NW
G = 8
GK = G * K
NG = TPW // G
D2 = D // 2
NC2 = D2 // L
NB = 2


def _sc_kernel(x_hbm, idx_hbm, w_hbm, tbl_hbm, out_hbm,
               idx_v, w_v, rows_v, x_v, o_v, sem_r, sem_x):
    wid = lax.axis_index("s") * 2 + lax.axis_index("c")
    t0 = wid * TPW

    pltpu.sync_copy(idx_hbm.at[pl.ds(t0 * K, TPW * K)], idx_v)
    pltpu.sync_copy(w_hbm.at[pl.ds(t0 * K, TPW * K)], w_v)

    def in_copies(g, b):
        return (
            pltpu.make_async_copy(
                x_hbm.at[pl.ds(t0 + g * G, G)], x_v.at[b], sem_x.at[b]),
            pltpu.make_async_copy(
                tbl_hbm.at[idx_v.at[pl.ds(g * GK, GK)]], rows_v.at[b],
                sem_r.at[b]),
        )

    def start_in(g, b):
        for c in in_copies(g, b):
            c.start()

    def wait_in(g, b):
        for c in in_copies(g, b):
            c.wait()

    start_in(0, 0)

    @pl.loop(0, NG, step=NB)
    def _group(g0):
        for b in range(NB):
            g = g0 + b
            nb = (b + 1) % NB

            @pl.when(g + 1 < NG)
            def _():
                start_in(g + 1, nb)

            wait_in(g, b)

            wvecs = [w_v[pl.ds(g * GK + j * L, L)] for j in range(GK // L)]

            def bfld(ref, idx):
                return plsc.bitcast(ref[idx], jnp.bfloat16)

            for i in range(G):
                def dot_body(c, accs, _i=i, _b=b):
                    s = pl.ds(c * L, L)
                    xc = bfld(x_v, (_b, _i, s))
                    return tuple(
                        accs[k] + xc * bfld(rows_v, (_b, _i * K + k, s))
                        for k in range(K)
                    )

                accs = lax.fori_loop(
                    0, NC2, dot_body,
                    tuple(jnp.zeros((2 * L,), jnp.bfloat16) for _ in range(K)),
                )

                weff = []
                for k in range(K):
                    lo, hi = plsc.unpack(
                        accs[k], format=plsc.PackFormat.INTERLEAVED)
                    p = jnp.sum(lo + hi)
                    pv = jnp.full((L,), p, jnp.float32)
                    e = jnp.exp(-2.0 * jnp.abs(pv))
                    th = jnp.sign(pv) * (1.0 - e) / (1.0 + e)
                    j = i * K + k
                    wf = th * wvecs[j // L][j % L]
                    weff.append(
                        plsc.pack(wf, wf, format=plsc.PackFormat.INTERLEAVED))

                @plsc.parallel_loop(0, D2, L, unroll=2)
                def _comb(cw, _i=i, _b=b, _weff=weff):
                    s = pl.ds(cw, L)
                    m = [_weff[k] * bfld(rows_v, (_b, _i * K + k, s))
                         for k in range(K)]
                    t0_, t1_ = m[0] + m[1], m[2] + m[3]
                    t2_, t3_ = m[4] + m[5], m[6] + m[7]
                    acc = (bfld(x_v, (_b, _i, s)) + (t0_ + t1_)) + (t2_ + t3_)
                    lo, hi = plsc.unpack(acc, format=plsc.PackFormat.INTERLEAVED)
                    o_v[_i, pl.ds(cw, L)] = lo
                    o_v[_i, pl.ds(D2 + cw, L)] = hi

            pltpu.sync_copy(o_v, out_hbm.at[pl.ds(t0 + g * G, G)])


def _bf16_as_i32(a):
    def rtne(f):
        u = lax.bitcast_convert_type(f, jnp.uint32)
        return (u + jnp.uint32(0x7FFF) + ((u >> 16) & jnp.uint32(1))) >> 16

    lob = rtne(a[:, :D2])
    hib = rtne(a[:, D2:])
    return lax.bitcast_convert_type(lob | (hib << 16), jnp.int32)


def _i32_as_f32(u):
    u = lax.bitcast_convert_type(u, jnp.uint32)
    lo = lax.bitcast_convert_type(u << 16, jnp.float32)
    hi = lax.bitcast_convert_type(u & jnp.uint32(0xFFFF0000), jnp.float32)
    return jnp.concatenate([lo, hi], axis=1)


def kernel(x, indices, weights, table):
    idx_flat = indices.astype(jnp.int32).reshape(-1)
    w_flat = weights.reshape(-1)
    x32 = _bf16_as_i32(x)
    tbl32 = _bf16_as_i32(table)
    mesh = plsc.VectorSubcoreMesh(core_axis_name="c", subcore_axis_name="s")
    cp = pltpu.CompilerParams()
    if "needs_layout_passes" in pltpu.CompilerParams.__dataclass_fields__:
        cp = dataclasses.replace(cp, needs_layout_passes=False)
    f = pl.kernel(
        _sc_kernel,
        mesh=mesh,
        compiler_params=cp,
        out_type=jax.ShapeDtypeStruct((TOKENS, D), jnp.float32),
        scratch_types=[
            pltpu.VMEM((TPW * K,), jnp.int32),
            pltpu.VMEM((TPW * K,), jnp.float32),
            pltpu.VMEM((NB, GK, D2), jnp.int32),
            pltpu.VMEM((NB, G, D2), jnp.int32),
            pltpu.VMEM((G, D), jnp.float32),
            pltpu.SemaphoreType.DMA((NB,)),
            pltpu.SemaphoreType.DMA((NB,)),
        ],
    )
    return f(x32, idx_flat, w_flat, tbl32)


# speedup vs baseline: 1.0249x; 1.0249x over previous
import dataclasses
import functools

import jax
import jax.numpy as jnp
from jax import lax
from jax.experimental import pallas as pl
from jax.experimental.pallas import tpu as pltpu
from jax.experimental.pallas import tpu_sc as plsc

TOKENS = 16384
D = 1024
K = 8
L = 16
NW = 32
TPW = TOKENS // NW
G = 8
GK = G * K
NG = TPW // G
D2 = D // 2
NC2 = D2 // L
NB = 2


def _sc_kernel(x_hbm, idx_hbm, w_hbm, tbl_hbm, out_hbm,
               idx_v, w_v, rows_v, x_v, o_v, sem_r, sem_x):
    wid = lax.axis_index("s") * 2 + lax.axis_index("c")
    t0 = wid * TPW

    pltpu.sync_copy(idx_hbm.at[pl.ds(t0 * K, TPW * K)], idx_v)
    pltpu.sync_copy(w_hbm.at[pl.ds(t0 * K, TPW * K)], w_v)

    def in_copies(g, b):
        return (
            pltpu.make_async_copy(
                x_hbm.at[pl.ds(t0 + g * G, G)], x_v.at[b], sem_x.at[b]),
            pltpu.make_async_copy(
                tbl_hbm.at[idx_v.at[pl.ds(g * GK, GK)]], rows_v.at[b],
                sem_r.at[b]),
        )

    def start_in(g, b):
        for c in in_copies(g, b):
            c.start()

    def wait_in(g, b):
        for c in in_copies(g, b):
            c.wait()

    start_in(0, 0)

    @pl.loop(0, NG, step=NB)
    def _group(g0):
        for b in range(NB):
            g = g0 + b
            nb = (b + 1) % NB

            @pl.when(g + 1 < NG)
            def _():
                start_in(g + 1, nb)

            wait_in(g, b)

            wvecs = [w_v[pl.ds(g * GK + j * L, L)] for j in range(GK // L)]

            def bfld(ref, idx):
                return plsc.bitcast(ref[idx], jnp.bfloat16)

            for i in range(G):
                zeros = tuple(
                    jnp.zeros((2 * L,), jnp.bfloat16) for _ in range(K))

                @plsc.parallel_loop(0, D2, L, carry=zeros)
                def accs(cw, accs_c, _i=i, _b=b):
                    s = pl.ds(cw, L)
                    xc = bfld(x_v, (_b, _i, s))
                    return tuple(
                        accs_c[k] + xc * bfld(rows_v, (_b, _i * K + k, s))
                        for k in range(K)
                    )

                weff = []
                for k in range(K):
                    lo, hi = plsc.unpack(
                        accs[k], format=plsc.PackFormat.INTERLEAVED)
                    p = jnp.sum(lo + hi)
                    pv = jnp.full((L,), p, jnp.float32)
                    e = jnp.exp(-2.0 * jnp.abs(pv))
                    th = jnp.sign(pv) * (1.0 - e) / (1.0 + e)
                    j = i * K + k
                    wf = th * wvecs[j // L][j % L]
                    weff.append(
                        plsc.pack(wf, wf, format=plsc.PackFormat.INTERLEAVED))

                @plsc.parallel_loop(0, D2, L)
                def _comb(cw, _i=i, _b=b, _weff=weff):
                    s = pl.ds(cw, L)
                    m = [_weff[k] * bfld(rows_v, (_b, _i * K + k, s))
                         for k in range(K)]
                    t0_, t1_ = m[0] + m[1], m[2] + m[3]
                    t2_, t3_ = m[4] + m[5], m[6] + m[7]
                    acc = (bfld(x_v, (_b, _i, s)) + (t0_ + t1_)) + (t2_ + t3_)
                    lo, hi = plsc.unpack(acc, format=plsc.PackFormat.INTERLEAVED)
                    o_v[_i, pl.ds(cw, L)] = lo
                    o_v[_i, pl.ds(D2 + cw, L)] = hi

            pltpu.sync_copy(o_v, out_hbm.at[pl.ds(t0 + g * G, G)])


def _bf16_as_i32(a):
    def rtne(f):
        u = lax.bitcast_convert_type(f, jnp.uint32)
        return (u + jnp.uint32(0x7FFF) + ((u >> 16) & jnp.uint32(1))) >> 16

    lob = rtne(a[:, :D2])
    hib = rtne(a[:, D2:])
    return lax.bitcast_convert_type(lob | (hib << 16), jnp.int32)


def _i32_as_f32(u):
    u = lax.bitcast_convert_type(u, jnp.uint32)
    lo = lax.bitcast_convert_type(u << 16, jnp.float32)
    hi = lax.bitcast_convert_type(u & jnp.uint32(0xFFFF0000), jnp.float32)
    return jnp.concatenate([lo, hi], axis=1)


def kernel(x, indices, weights, table):
    idx_flat = indices.astype(jnp.int32).reshape(-1)
    w_flat = weights.reshape(-1)
    x32 = _bf16_as_i32(x)
    tbl32 = _bf16_as_i32(table)
    mesh = plsc.VectorSubcoreMesh(core_axis_name="c", subcore_axis_name="s")
    cp = pltpu.CompilerParams()
    if "needs_layout_passes" in pltpu.CompilerParams.__dataclass_fields__:
        cp = dataclasses.replace(cp, needs_layout_passes=False)
    f = pl.kernel(
        _sc_kernel,
        mesh=mesh,
        compiler_params=cp,
        out_type=jax.ShapeDtypeStruct((TOKENS, D), jnp.float32),
        scratch_types=[
            pltpu.VMEM((TPW * K,), jnp.int32),
            pltpu.VMEM((TPW * K,), jnp.float32),
            pltpu.VMEM((NB, GK, D2), jnp.int32),
            pltpu.VMEM((NB, G, D2), jnp.int32),
            pltpu.VMEM((G, D), jnp.float32),
            pltpu.SemaphoreType.DMA((NB,)),
            pltpu.SemaphoreType.DMA((NB,)),
        ],
    )
    return f(x32, idx_flat, w_flat, tbl32)


# speedup vs baseline: 1.0348x; 1.0096x over previous
import dataclasses

import jax
import jax.numpy as jnp
from jax import lax
from jax.experimental import pallas as pl
from jax.experimental.pallas import tpu as pltpu
from jax.experimental.pallas import tpu_sc as plsc

TOKENS = 16384
D = 1024
K = 8
L = 16
NW = 32
TPW = TOKENS // NW
G = 8
GK = G * K
NG = TPW // G
D2 = D // 2
NC2 = D2 // L
NB = 2


def _sc_kernel(x_hbm, idx_hbm, w_hbm, tbl_hbm, out_hbm,
               idx_v, w_v, rows_v, x_v, o_v, sem_r, sem_x):
    wid = lax.axis_index("s") * 2 + lax.axis_index("c")
    t0 = wid * TPW

    pltpu.sync_copy(idx_hbm.at[pl.ds(t0 * K, TPW * K)], idx_v)
    pltpu.sync_copy(w_hbm.at[pl.ds(t0 * K, TPW * K)], w_v)

    def in_copies(g, b):
        return (
            pltpu.make_async_copy(
                x_hbm.at[pl.ds(t0 + g * G, G)], x_v.at[b], sem_x.at[b]),
            pltpu.make_async_copy(
                tbl_hbm.at[idx_v.at[pl.ds(g * GK, GK)]], rows_v.at[b],
                sem_r.at[b]),
        )

    def start_in(g, b):
        for c in in_copies(g, b):
            c.start()

    def wait_in(g, b):
        for c in in_copies(g, b):
            c.wait()

    start_in(0, 0)

    @pl.loop(0, NG, step=NB)
    def _group(g0):
        for b in range(NB):
            g = g0 + b
            nb = (b + 1) % NB

            @pl.when(g + 1 < NG)
            def _():
                start_in(g + 1, nb)

            wait_in(g, b)

            wvecs = [w_v[pl.ds(g * GK + j * L, L)] for j in range(GK // L)]

            def bfld(ref, idx):
                return plsc.bitcast(ref[idx], jnp.bfloat16)

            for i in range(G):
                def dot_body(c, accs, _i=i, _b=b):
                    s = pl.ds(c * L, L)
                    xc = bfld(x_v, (_b, _i, s))
                    return tuple(
                        accs[k] + xc * bfld(rows_v, (_b, _i * K + k, s))
                        for k in range(K)
                    )

                accs = lax.fori_loop(
                    0, NC2, dot_body,
                    tuple(jnp.zeros((2 * L,), jnp.bfloat16) for _ in range(K)),
                )

                weff = []
                for k in range(K):
                    lo, hi = plsc.unpack(
                        accs[k], format=plsc.PackFormat.INTERLEAVED)
                    p = jnp.sum(lo + hi)
                    pv = jnp.full((L,), p, jnp.float32)
                    e = jnp.exp(-2.0 * jnp.abs(pv))
                    th = jnp.sign(pv) * (1.0 - e) / (1.0 + e)
                    j = i * K + k
                    wf = th * wvecs[j // L][j % L]
                    weff.append(
                        plsc.pack(wf, wf, format=plsc.PackFormat.INTERLEAVED))

                @plsc.parallel_loop(0, D2, L)
                def _comb(cw, _i=i, _b=b, _weff=weff):
                    s = pl.ds(cw, L)
                    m = [_weff[k] * bfld(rows_v, (_b, _i * K + k, s))
                         for k in range(K)]
                    t0_, t1_ = m[0] + m[1], m[2] + m[3]
                    t2_, t3_ = m[4] + m[5], m[6] + m[7]
                    acc = (bfld(x_v, (_b, _i, s)) + (t0_ + t1_)) + (t2_ + t3_)
                    lo, hi = plsc.unpack(acc, format=plsc.PackFormat.INTERLEAVED)
                    o_v[_i, pl.ds(cw, L)] = lo
                    o_v[_i, pl.ds(D2 + cw, L)] = hi

            pltpu.sync_copy(o_v, out_hbm.at[pl.ds(t0 + g * G, G)])


def _bf16_as_i32(a):
    def rtne(f):
        u = lax.bitcast_convert_type(f, jnp.uint32)
        return (u + jnp.uint32(0x7FFF) + ((u >> 16) & jnp.uint32(1))) >> 16

    lob = rtne(a[:, :D2])
    hib = rtne(a[:, D2:])
    return lax.bitcast_convert_type(lob | (hib << 16), jnp.int32)


def kernel(x, indices, weights, table):
    idx_flat = indices.astype(jnp.int32).reshape(-1)
    w_flat = weights.reshape(-1)
    x32 = _bf16_as_i32(x)
    tbl32 = _bf16_as_i32(table)
    mesh = plsc.VectorSubcoreMesh(core_axis_name="c", subcore_axis_name="s")
    cp = pltpu.CompilerParams()
    if "needs_layout_passes" in pltpu.CompilerParams.__dataclass_fields__:
        cp = dataclasses.replace(cp, needs_layout_passes=False)
    f = pl.kernel(
        _sc_kernel,
        mesh=mesh,
        compiler_params=cp,
        out_type=jax.ShapeDtypeStruct((TOKENS, D), jnp.float32),
        scratch_types=[
            pltpu.VMEM((TPW * K,), jnp.int32),
            pltpu.VMEM((TPW * K,), jnp.float32),
            pltpu.VMEM((NB, GK, D2), jnp.int32),
            pltpu.VMEM((NB, G, D2), jnp.int32),
            pltpu.VMEM((G, D), jnp.float32),
            pltpu.SemaphoreType.DMA((NB,)),
            pltpu.SemaphoreType.DMA((NB,)),
        ],
    )
    return f(x32, idx_flat, w_flat, tbl32)
